# baseline (speedup 1.0000x reference)
import functools

import jax
import jax.numpy as jnp
from jax import lax
from jax.experimental import pallas as pl
from jax.experimental.pallas import tpu as pltpu
from jax.experimental.pallas import tpu_sc as plsc

_B = 4096
_SEQ = 50
_D = 128
_NC = 2
_NS = 16
_NW = _NC * _NS
_BPW = _B // _NW
_CHUNK = 8
_STEPS = _BPW // _CHUNK
_G = _CHUNK * _SEQ
_NLANE = _D // 16
_KOFF = 2


def _gather_mean(idx_flat, table):
    mesh = plsc.VectorSubcoreMesh(core_axis_name="c", subcore_axis_name="s")

    @functools.partial(
        pl.kernel,
        mesh=mesh,
        out_type=jax.ShapeDtypeStruct((_B, _D), jnp.float32),
        scratch_types=[
            pltpu.VMEM((_BPW * _SEQ,), jnp.int32),
            pltpu.VMEM((_G, _D), jnp.float32),
            pltpu.VMEM((_G, _D), jnp.float32),
            pltpu.VMEM((_CHUNK, _D), jnp.float32),
            pltpu.VMEM((_CHUNK, _D), jnp.float32),
            pltpu.VMEM((_KOFF, _D), jnp.float32),
            pltpu.VMEM((_KOFF, _D), jnp.float32),
            pltpu.VMEM_SHARED((_NS * _KOFF, _D), jnp.float32),
            pltpu.SemaphoreType.DMA,
            pltpu.SemaphoreType.DMA,
            pltpu.SemaphoreType.DMA,
            pltpu.SemaphoreType.DMA,
            pltpu.SemaphoreType.DMA,
        ],
    )
    def k(idx_hbm, table_hbm, out_hbm, idx_all, rows0, rows1, pooled0,
          pooled1, zeros_v, accrd_v, shacc, sem0, sem1, semo0, semo1, sems):
        wid = lax.axis_index("s") * _NC + lax.axis_index("c")
        sid = lax.axis_index("s")
        base = wid * _BPW

        for kk in range(_KOFF):
            for c in range(_NLANE):
                zeros_v[kk, pl.ds(c * 16, 16)] = jnp.zeros((16,), jnp.float32)

        pltpu.sync_copy(idx_hbm.at[pl.ds(base * _SEQ, _BPW * _SEQ)], idx_all)

        def issue(s, rows_v, sem):
            pltpu.async_copy(
                table_hbm.at[idx_all.at[pl.ds(s * _G, _G)]], rows_v, sem)

        def wait(s, rows_v, sem):
            pltpu.make_async_copy(
                table_hbm.at[idx_all.at[pl.ds(s * _G, _G)]], rows_v, sem
            ).wait()

        def out_slot(s):
            return out_hbm.at[pl.ds(base + s * _CHUNK, _CHUNK)]

        _NSC = (_SEQ // 16) * 16

        def offload_descr(kk, g, rows_v):
            r = _CHUNK - _KOFF + kk
            dest = jnp.full((16,), sid * _KOFF + kk, jnp.int32)
            src = rows_v.at[pl.ds(r * _SEQ + g * 16, 16)]
            return src, shacc.at[dest]

        def reduce_store(s, rows_v, pooled_v, semo):
            pltpu.sync_copy(zeros_v, shacc.at[pl.ds(sid * _KOFF, _KOFF)])
            for kk in range(_KOFF):
                for g in range(_NSC // 16):
                    src, dst = offload_descr(kk, g, rows_v)
                    pltpu.async_copy(src, dst, sems, add=True)
            for r in range(_CHUNK - _KOFF):
                def red(j, acc):
                    return tuple(
                        acc[c] + rows_v[r * _SEQ + j, pl.ds(c * 16, 16)]
                        for c in range(_NLANE)
                    )
                acc0 = tuple(jnp.zeros((16,), jnp.float32) for _ in range(_NLANE))
                acc = lax.fori_loop(0, _SEQ, red, acc0)
                for c in range(_NLANE):
                    pooled_v[r, pl.ds(c * 16, 16)] = acc[c] * (1.0 / _SEQ)
            for kk in range(_KOFF):
                for g in range(_NSC // 16):
                    src, dst = offload_descr(kk, g, rows_v)
                    pltpu.make_async_copy(src, dst, sems).wait()
            pltpu.sync_copy(shacc.at[pl.ds(sid * _KOFF, _KOFF)], accrd_v)
            for kk in range(_KOFF):
                r = _CHUNK - _KOFF + kk
                for c in range(_NLANE):
                    acc = accrd_v[kk, pl.ds(c * 16, 16)]
                    for j in range(_NSC, _SEQ):
                        acc = acc + rows_v[r * _SEQ + j, pl.ds(c * 16, 16)]
                    pooled_v[r, pl.ds(c * 16, 16)] = acc * (1.0 / _SEQ)
            pltpu.async_copy(pooled_v, out_slot(s), semo)

        def wait_out(s, pooled_v, semo):
            pltpu.make_async_copy(pooled_v, out_slot(s), semo).wait()

        issue(0, rows0, sem0)

        def pair(i, carry):
            s = 2 * i
            issue(s + 1, rows1, sem1)
            wait(s, rows0, sem0)

            @pl.when(s >= 2)
            def _():
                wait_out(lax.max(s - 2, 0), pooled0, semo0)

            reduce_store(s, rows0, pooled0, semo0)

            @pl.when(s + 2 < _STEPS)
            def _():
                issue(s + 2, rows0, sem0)

            wait(s + 1, rows1, sem1)

            @pl.when(s >= 2)
            def _():
                wait_out(lax.max(s - 1, 0), pooled1, semo1)

            reduce_store(s + 1, rows1, pooled1, semo1)
            return carry

        lax.fori_loop(0, _STEPS // 2, pair, 0)
        wait_out(_STEPS - 2, pooled0, semo0)
        wait_out(_STEPS - 1, pooled1, semo1)

    return k(idx_flat, table)


def _linear(pooled, W, b):
    def mm(p_ref, w_ref, b_ref, o_ref):
        o_ref[...] = lax.dot_general(
            p_ref[...], w_ref[...], (((1,), (1,)), ((), ())),
            preferred_element_type=jnp.float32,
        ) + b_ref[...]

    return pl.pallas_call(
        mm,
        out_shape=jax.ShapeDtypeStruct((_B, _D), jnp.float32),
    )(pooled, W, b.reshape(1, _D))


def kernel(indices, table, W, b):
    idx_flat = indices.reshape(-1).astype(jnp.int32)
    pooled = _gather_mean(idx_flat, table)
    return _linear(pooled, W, b)


# speedup vs baseline: 1.0374x; 1.0374x over previous
import functools

import jax
import jax.numpy as jnp
from jax import lax
from jax.experimental import pallas as pl
from jax.experimental.pallas import tpu as pltpu
from jax.experimental.pallas import tpu_sc as plsc

_B = 4096
_SEQ = 50
_D = 128
_NC = 2
_NS = 16
_NW = _NC * _NS
_BPW = _B // _NW
_CHUNK = 8
_STEPS = _BPW // _CHUNK
_G = _CHUNK * _SEQ
_NLANE = _D // 16


def _gather_mean(idx_flat, table):
    mesh = plsc.VectorSubcoreMesh(core_axis_name="c", subcore_axis_name="s")

    @functools.partial(
        pl.kernel,
        mesh=mesh,
        out_type=jax.ShapeDtypeStruct((_B, _D), jnp.float32),
        scratch_types=[
            pltpu.VMEM((_BPW * _SEQ,), jnp.int32),
            pltpu.VMEM((_G, _D), jnp.float32),
            pltpu.VMEM((_G, _D), jnp.float32),
            pltpu.VMEM((_CHUNK, _D), jnp.float32),
            pltpu.VMEM((_CHUNK, _D), jnp.float32),
            pltpu.SemaphoreType.DMA,
            pltpu.SemaphoreType.DMA,
            pltpu.SemaphoreType.DMA,
            pltpu.SemaphoreType.DMA,
        ],
    )
    def k(idx_hbm, table_hbm, out_hbm, idx_all, rows0, rows1, pooled0,
          pooled1, sem0, sem1, semo0, semo1):
        wid = lax.axis_index("s") * _NC + lax.axis_index("c")
        base = wid * _BPW

        pltpu.sync_copy(idx_hbm.at[pl.ds(base * _SEQ, _BPW * _SEQ)], idx_all)

        def issue(s, rows_v, sem):
            pltpu.async_copy(
                table_hbm.at[idx_all.at[pl.ds(s * _G, _G)]], rows_v, sem)

        def wait(s, rows_v, sem):
            pltpu.make_async_copy(
                table_hbm.at[idx_all.at[pl.ds(s * _G, _G)]], rows_v, sem
            ).wait()

        def out_slot(s):
            return out_hbm.at[pl.ds(base + s * _CHUNK, _CHUNK)]

        def reduce_store(s, rows_v, pooled_v, semo):
            for r in range(_CHUNK):
                def red(j, acc):
                    return tuple(
                        acc[c] + rows_v[r * _SEQ + j, pl.ds(c * 16, 16)]
                        for c in range(_NLANE)
                    )
                acc0 = tuple(jnp.zeros((16,), jnp.float32) for _ in range(_NLANE))
                acc = lax.fori_loop(0, _SEQ, red, acc0)
                for c in range(_NLANE):
                    pooled_v[r, pl.ds(c * 16, 16)] = acc[c] * (1.0 / _SEQ)
            pltpu.async_copy(pooled_v, out_slot(s), semo)

        def wait_out(s, pooled_v, semo):
            pltpu.make_async_copy(pooled_v, out_slot(s), semo).wait()

        issue(0, rows0, sem0)

        def pair(i, carry):
            s = 2 * i
            issue(s + 1, rows1, sem1)
            wait(s, rows0, sem0)

            @pl.when(s >= 2)
            def _():
                wait_out(lax.max(s - 2, 0), pooled0, semo0)

            reduce_store(s, rows0, pooled0, semo0)

            @pl.when(s + 2 < _STEPS)
            def _():
                issue(s + 2, rows0, sem0)

            wait(s + 1, rows1, sem1)

            @pl.when(s >= 2)
            def _():
                wait_out(lax.max(s - 1, 0), pooled1, semo1)

            reduce_store(s + 1, rows1, pooled1, semo1)
            return carry

        lax.fori_loop(0, _STEPS // 2, pair, 0)
        wait_out(_STEPS - 2, pooled0, semo0)
        wait_out(_STEPS - 1, pooled1, semo1)

    return k(idx_flat, table)


def _linear(pooled, W, b):
    def mm(p_ref, w_ref, b_ref, o_ref):
        o_ref[...] = lax.dot_general(
            p_ref[...], w_ref[...], (((1,), (1,)), ((), ())),
            preferred_element_type=jnp.float32,
        ) + b_ref[...]

    return pl.pallas_call(
        mm,
        out_shape=jax.ShapeDtypeStruct((_B, _D), jnp.float32),
    )(pooled, W, b.reshape(1, _D))


def kernel(indices, table, W, b):
    idx_flat = indices.reshape(-1).astype(jnp.int32)
    pooled = _gather_mean(idx_flat, table)
    return _linear(pooled, W, b)
